# baseline (speedup 1.0000x reference)
import functools
import jax
import jax.numpy as jnp
from jax.experimental import pallas as pl

N_BLOCK = 1000


def _embed_kernel(batch_ref, attrs_ref, feats_ref, fieldx_ref, ew_ref, out_ref):
    B = batch_ref.shape[-1]
    G = fieldx_ref.shape[0]
    attrs = attrs_ref[0]
    mx = jnp.max(attrs, axis=0, keepdims=True)
    row_ids = jax.lax.broadcasted_iota(jnp.int32, attrs.shape, 0)
    idx = jnp.min(jnp.where(attrs == mx, row_ids, 127), axis=0, keepdims=True)
    s = jnp.zeros((1, B), jnp.float32)
    for e in range(5):
        s = jnp.where(idx == e, ew_ref[0, e], s)
    b = batch_ref[0]
    g_ids = jax.lax.broadcasted_iota(jnp.int32, (G, B), 0)
    oh = jnp.where(g_ids == b, s, 0.0)
    mult = jax.lax.dot_general(
        oh, fieldx_ref[...], (((0,), (0,)), ((), ())),
        preferred_element_type=jnp.float32)
    out_ref[...] = feats_ref[...] + mult


@jax.jit
def kernel(batch, node_feats_1, node_attrs, data_external_field,
           element_weights, channel_weights):
    N, C, K = node_feats_1.shape
    G = data_external_field.shape[0]
    E = node_attrs.shape[1]
    feats = node_feats_1.reshape(N, C * K)
    batch_r = batch.astype(jnp.int32).reshape(N // N_BLOCK, 1, N_BLOCK)
    attrs_t = jnp.pad(node_attrs.T, ((0, 8 - E), (0, 0)),
                      constant_values=-jnp.inf)
    attrs_3d = attrs_t.reshape(8, N // N_BLOCK, N_BLOCK).transpose(1, 0, 2)
    fieldx = (channel_weights[None, :, None]
              * data_external_field[:, None, :]).reshape(G, C * K)
    ew_pad = jnp.zeros((1, 128), jnp.float32).at[0, :E].set(element_weights)

    grid = (N // N_BLOCK,)
    out = pl.pallas_call(
        _embed_kernel,
        grid=grid,
        in_specs=[
            pl.BlockSpec((1, 1, N_BLOCK), lambda i: (i, 0, 0)),
            pl.BlockSpec((1, 8, N_BLOCK), lambda i: (i, 0, 0)),
            pl.BlockSpec((N_BLOCK, C * K), lambda i: (i, 0)),
            pl.BlockSpec((G, C * K), lambda i: (0, 0)),
            pl.BlockSpec((1, 128), lambda i: (0, 0)),
        ],
        out_specs=pl.BlockSpec((N_BLOCK, C * K), lambda i: (i, 0)),
        out_shape=jax.ShapeDtypeStruct((N, C * K), jnp.float32),
    )(batch_r, attrs_3d, feats, fieldx, ew_pad)
    return out.reshape(N, C, K)
